# Initial kernel scaffold; baseline (speedup 1.0000x reference)
import jax
import jax.numpy as jnp
from jax.experimental import pallas as pl


def kernel(inputs, W_word, P_pos):
    raise NotImplementedError("write your pallas kernel here")


# speedup vs baseline: 1.3389x; 1.3389x over previous
import functools

import jax
import jax.numpy as jnp
from jax import lax
from jax.experimental import pallas as pl
from jax.experimental.pallas import tpu as pltpu
from jax.experimental.pallas import tpu_sc as plsc

NUM_CORES = 2
NUM_SUBCORES = 16
NW = NUM_CORES * NUM_SUBCORES
CHUNK = 128


def _emb_kernel(n_tokens, dim, n_per_w):
    k = n_per_w // CHUNK
    mesh = plsc.VectorSubcoreMesh(core_axis_name="c", subcore_axis_name="s")

    @functools.partial(
        pl.kernel,
        mesh=mesh,
        out_type=jax.ShapeDtypeStruct((n_tokens, dim), jnp.float32),
        scratch_types=[
            pltpu.VMEM((k, CHUNK), jnp.int32),
            pltpu.VMEM((n_per_w, dim), jnp.float32),
            pltpu.SemaphoreType.DMA,
        ],
    )
    def emb(idx_hbm, table_hbm, pos_hbm, out_hbm, idx_v, rows_v, sem):
        wid = lax.axis_index("s") * NUM_CORES + lax.axis_index("c")
        base = wid * n_per_w
        pos_base = lax.rem(base, pos_hbm.shape[0])
        pltpu.sync_copy(pos_hbm.at[pl.ds(pos_base, n_per_w)], rows_v)
        pltpu.sync_copy(idx_hbm.at[pl.ds(wid * k, k)], idx_v)
        copies = [
            pltpu.async_copy(
                table_hbm.at[idx_v.at[j]],
                rows_v.at[pl.ds(j * CHUNK, CHUNK)],
                sem,
                add=True,
            )
            for j in range(k)
        ]
        for c in copies:
            c.wait()
        pltpu.sync_copy(rows_v, out_hbm.at[pl.ds(base, n_per_w)])

    return emb


def kernel(inputs, W_word, P_pos):
    batch, seq_len = inputs.shape
    vocab, dim = W_word.shape
    n_tokens = batch * seq_len
    n_per_w = n_tokens // NW
    idx2d = inputs.reshape(n_tokens // CHUNK, CHUNK)
    out = _emb_kernel(n_tokens, dim, n_per_w)(idx2d, W_word, P_pos)
    return out.reshape(batch, seq_len, dim)


# speedup vs baseline: 1.3803x; 1.0309x over previous
import functools

import jax
import jax.numpy as jnp
from jax import lax
from jax.experimental import pallas as pl
from jax.experimental.pallas import tpu as pltpu
from jax.experimental.pallas import tpu_sc as plsc

NUM_CORES = 2
NUM_SUBCORES = 16
NW = NUM_CORES * NUM_SUBCORES


def _emb_kernel(batch, seq_len, vocab, dim):
    s_per_w = seq_len // NW
    mesh = plsc.VectorSubcoreMesh(core_axis_name="c", subcore_axis_name="s")

    @functools.partial(
        pl.kernel,
        mesh=mesh,
        out_type=jax.ShapeDtypeStruct((batch, seq_len, dim), jnp.float32),
        scratch_types=[
            pltpu.VMEM((batch, s_per_w), jnp.int32),
            pltpu.VMEM((batch, s_per_w, dim), jnp.float32),
            pltpu.SemaphoreType.DMA,
            pltpu.SemaphoreType.DMA,
            pltpu.SemaphoreType.DMA((batch,)),
            pltpu.SemaphoreType.DMA,
        ],
    )
    def emb(idx_hbm, table_hbm, pos_hbm, out_hbm, idx_v, rows_v, sem_p, sem_i,
            sem_g, sem_o):
        wid = lax.axis_index("s") * NUM_CORES + lax.axis_index("c")
        base = wid * s_per_w
        pos_cps = [
            pltpu.async_copy(pos_hbm.at[pl.ds(base, s_per_w)], rows_v.at[b],
                             sem_p)
            for b in range(batch)
        ]
        cp_idx = pltpu.async_copy(idx_hbm.at[wid], idx_v, sem_i)
        for c in pos_cps:
            c.wait()
        cp_idx.wait()
        gathers = [
            pltpu.async_copy(table_hbm.at[idx_v.at[b]], rows_v.at[b],
                             sem_g.at[b], add=True)
            for b in range(batch)
        ]
        outs = []
        for b in range(batch):
            gathers[b].wait()
            outs.append(
                pltpu.async_copy(rows_v.at[b],
                                 out_hbm.at[b, pl.ds(base, s_per_w)], sem_o))
        for c in outs:
            c.wait()

    return emb


def kernel(inputs, W_word, P_pos):
    batch, seq_len = inputs.shape
    vocab, dim = W_word.shape
    s_per_w = seq_len // NW
    idx3d = inputs.reshape(batch, NW, s_per_w).transpose(1, 0, 2)
    return _emb_kernel(batch, seq_len, vocab, dim)(idx3d, W_word, P_pos)
